# Initial kernel scaffold; baseline (speedup 1.0000x reference)
import jax
import jax.numpy as jnp
from jax.experimental import pallas as pl


def kernel(x, img_idx, white_balance_parameter):
    raise NotImplementedError("write your pallas kernel here")


# speedup vs baseline: 2.1216x; 2.1216x over previous
import functools

import jax
import jax.numpy as jnp
from jax import lax
from jax.experimental import pallas as pl
from jax.experimental.pallas import tpu as pltpu
from jax.experimental.pallas import tpu_sc as plsc

_L = 16
_NW = 32


def kernel(x, img_idx, white_balance_parameter):
    n = x.shape[0]
    x_flat = x.reshape(-1)
    idx_flat = img_idx.astype(jnp.int32).reshape(-1)
    tbl_flat = white_balance_parameter.reshape(-1)
    n_rows = white_balance_parameter.shape[0]

    per_w = n // _NW
    chunk = 4096
    n_chunks = per_w // chunk

    mesh = plsc.VectorSubcoreMesh(core_axis_name="c", subcore_axis_name="s")

    lane_consts = jnp.array(
        [(_L * k + l) // 3 for k in range(3) for l in range(_L)]
        + [(_L * k + l) % 3 for k in range(3) for l in range(_L)],
        dtype=jnp.int32,
    )

    @functools.partial(
        pl.kernel,
        mesh=mesh,
        compiler_params=pltpu.CompilerParams(needs_layout_passes=False),
        out_type=jax.ShapeDtypeStruct((3 * n,), jnp.float32),
        scratch_types=[
            pltpu.VMEM((3 * n_rows,), jnp.float32),
            pltpu.VMEM((6 * _L,), jnp.int32),
            pltpu.VMEM((chunk,), jnp.int32),
            pltpu.VMEM((3 * chunk,), jnp.float32),
            pltpu.VMEM((3 * chunk,), jnp.float32),
        ],
    )
    def wb(x_hbm, idx_hbm, tbl_hbm, lc_hbm, out_hbm, tbl_v, lc_v, idx_v, x_v, o_v):
        wid = lax.axis_index("s") * 2 + lax.axis_index("c")
        ray0 = wid * per_w
        pltpu.sync_copy(tbl_hbm, tbl_v)
        pltpu.sync_copy(lc_hbm, lc_v)

        rayoffs = [lc_v[pl.ds(_L * k, _L)] for k in range(3)]
        chs = [lc_v[pl.ds(_L * (3 + k), _L)] for k in range(3)]

        def chunk_body(cidx, carry):
            rbase = ray0 + cidx * chunk
            pltpu.sync_copy(idx_hbm.at[pl.ds(rbase, chunk)], idx_v)
            pltpu.sync_copy(x_hbm.at[pl.ds(3 * rbase, 3 * chunk)], x_v)

            def group_body(g, c2):
                for k in range(3):
                    rows = plsc.load_gather(idx_v, [_L * g + rayoffs[k]])
                    tv = plsc.load_gather(tbl_v, [3 * rows + chs[k]])
                    base = 48 * g + _L * k
                    o_v[pl.ds(base, _L)] = x_v[pl.ds(base, _L)] * tv
                return c2

            lax.fori_loop(0, chunk // _L, group_body, 0)
            pltpu.sync_copy(o_v, out_hbm.at[pl.ds(3 * rbase, 3 * chunk)])
            return carry

        lax.fori_loop(0, n_chunks, chunk_body, 0)

    out_flat = wb(x_flat, idx_flat, tbl_flat, lane_consts)
    return out_flat.reshape(n, 3)


# speedup vs baseline: 8.5874x; 4.0476x over previous
import functools

import jax
import jax.numpy as jnp
from jax import lax
from jax.experimental import pallas as pl
from jax.experimental.pallas import tpu as pltpu
from jax.experimental.pallas import tpu_sc as plsc

_L = 16
_NW = 32


def kernel(x, img_idx, white_balance_parameter):
    n = x.shape[0]
    n_rows = white_balance_parameter.shape[0]
    x_cm = x.T.reshape(3 * n)
    idx_1d = img_idx.astype(jnp.int32).reshape(n)
    tbl_cm = white_balance_parameter.T.reshape(3 * n_rows)

    per_w = n // _NW
    chunk = 4096
    n_chunks = per_w // chunk

    mesh = plsc.VectorSubcoreMesh(core_axis_name="c", subcore_axis_name="s")

    @functools.partial(
        pl.kernel,
        mesh=mesh,
        compiler_params=pltpu.CompilerParams(needs_layout_passes=False),
        out_type=jax.ShapeDtypeStruct((3 * n,), jnp.float32),
        scratch_types=[
            pltpu.VMEM((n_rows,), jnp.float32),
            pltpu.VMEM((n_rows,), jnp.float32),
            pltpu.VMEM((n_rows,), jnp.float32),
            pltpu.VMEM((chunk,), jnp.int32),
            pltpu.VMEM((3 * chunk,), jnp.float32),
            pltpu.VMEM((3 * chunk,), jnp.float32),
        ],
    )
    def wb(x_hbm, idx_hbm, tbl_hbm, out_hbm, t0_v, t1_v, t2_v, idx_v, x_v, o_v):
        wid = lax.axis_index("s") * 2 + lax.axis_index("c")
        ray0 = wid * per_w
        tbls = (t0_v, t1_v, t2_v)
        for c in range(3):
            pltpu.sync_copy(tbl_hbm.at[pl.ds(c * n_rows, n_rows)], tbls[c])

        def chunk_body(cidx, carry):
            rbase = ray0 + cidx * chunk
            pltpu.sync_copy(idx_hbm.at[pl.ds(rbase, chunk)], idx_v)
            for c in range(3):
                pltpu.sync_copy(
                    x_hbm.at[pl.ds(c * n + rbase, chunk)],
                    x_v.at[pl.ds(c * chunk, chunk)],
                )

            def group_body(g, c2):
                idxv = idx_v[pl.ds(_L * g, _L)]
                for c in range(3):
                    tv = plsc.load_gather(tbls[c], [idxv])
                    base = c * chunk + _L * g
                    o_v[pl.ds(base, _L)] = x_v[pl.ds(base, _L)] * tv
                return c2

            lax.fori_loop(0, chunk // _L, group_body, 0)
            for c in range(3):
                pltpu.sync_copy(
                    o_v.at[pl.ds(c * chunk, chunk)],
                    out_hbm.at[pl.ds(c * n + rbase, chunk)],
                )
            return carry

        lax.fori_loop(0, n_chunks, chunk_body, 0)

    out_cm = wb(x_cm, idx_1d, tbl_cm)
    return out_cm.reshape(3, n).T


# speedup vs baseline: 9.0582x; 1.0548x over previous
import functools

import jax
import jax.numpy as jnp
from jax import lax
from jax.experimental import pallas as pl
from jax.experimental.pallas import tpu as pltpu
from jax.experimental.pallas import tpu_sc as plsc

_L = 16
_NW = 32


def kernel(x, img_idx, white_balance_parameter):
    n = x.shape[0]
    n_rows = white_balance_parameter.shape[0]
    x_cm = x.T.reshape(3 * n)
    idx_1d = img_idx.astype(jnp.int32).reshape(n)
    tbl_cm = white_balance_parameter.T.reshape(3 * n_rows)

    per_w = n // _NW
    chunk = 8192
    n_chunks = per_w // chunk

    mesh = plsc.VectorSubcoreMesh(core_axis_name="c", subcore_axis_name="s")

    @functools.partial(
        pl.kernel,
        mesh=mesh,
        compiler_params=pltpu.CompilerParams(needs_layout_passes=False),
        out_type=jax.ShapeDtypeStruct((3 * n,), jnp.float32),
        scratch_types=[
            pltpu.VMEM((n_rows,), jnp.float32),
            pltpu.VMEM((n_rows,), jnp.float32),
            pltpu.VMEM((n_rows,), jnp.float32),
            pltpu.VMEM((chunk,), jnp.int32),
            pltpu.VMEM((chunk,), jnp.int32),
            pltpu.VMEM((3 * chunk,), jnp.float32),
            pltpu.VMEM((3 * chunk,), jnp.float32),
            pltpu.VMEM((3 * chunk,), jnp.float32),
            pltpu.VMEM((3 * chunk,), jnp.float32),
            pltpu.SemaphoreType.DMA,
            pltpu.SemaphoreType.DMA,
            pltpu.SemaphoreType.DMA,
            pltpu.SemaphoreType.DMA,
        ],
    )
    def wb(
        x_hbm, idx_hbm, tbl_hbm, out_hbm,
        t0_v, t1_v, t2_v, idx0_v, idx1_v, x0_v, x1_v, o0_v, o1_v,
        sem_in0, sem_in1, sem_out0, sem_out1,
    ):
        wid = lax.axis_index("s") * 2 + lax.axis_index("c")
        ray0 = wid * per_w
        tbls = (t0_v, t1_v, t2_v)
        idx_bufs = (idx0_v, idx1_v)
        x_bufs = (x0_v, x1_v)
        o_bufs = (o0_v, o1_v)
        sems_in = (sem_in0, sem_in1)
        sems_out = (sem_out0, sem_out1)
        for c in range(3):
            pltpu.sync_copy(tbl_hbm.at[pl.ds(c * n_rows, n_rows)], tbls[c])

        def in_copies(p, cidx):
            rbase = ray0 + cidx * chunk
            copies = [
                pltpu.make_async_copy(
                    idx_hbm.at[pl.ds(rbase, chunk)], idx_bufs[p], sems_in[p]
                )
            ]
            for c in range(3):
                copies.append(
                    pltpu.make_async_copy(
                        x_hbm.at[pl.ds(c * n + rbase, chunk)],
                        x_bufs[p].at[pl.ds(c * chunk, chunk)],
                        sems_in[p],
                    )
                )
            return copies

        def out_copies(p, cidx):
            rbase = ray0 + cidx * chunk
            return [
                pltpu.make_async_copy(
                    o_bufs[p].at[pl.ds(c * chunk, chunk)],
                    out_hbm.at[pl.ds(c * n + rbase, chunk)],
                    sems_out[p],
                )
                for c in range(3)
            ]

        for cp in in_copies(0, 0):
            cp.start()

        def do_chunk(p, cidx):
            @pl.when(cidx + 1 < n_chunks)
            def _():
                for cp in in_copies(1 - p, cidx + 1):
                    cp.start()

            for cp in in_copies(p, cidx):
                cp.wait()

            @pl.when(cidx >= 2)
            def _():
                for cp in out_copies(p, cidx - 2):
                    cp.wait()

            def group_body(g, c2):
                idxv = idx_bufs[p][pl.ds(_L * g, _L)]
                for c in range(3):
                    tv = plsc.load_gather(tbls[c], [idxv])
                    base = c * chunk + _L * g
                    o_bufs[p][pl.ds(base, _L)] = x_bufs[p][pl.ds(base, _L)] * tv
                return c2

            lax.fori_loop(0, chunk // _L, group_body, 0, unroll=8)

            for cp in out_copies(p, cidx):
                cp.start()

        def loop_body(base_cidx, carry):
            do_chunk(0, base_cidx)
            do_chunk(1, base_cidx + 1)
            return carry

        lax.fori_loop(0, n_chunks // 2, lambda i, c: loop_body(2 * i, c), 0)

        for p, cidx in ((0, n_chunks - 2), (1, n_chunks - 1)):
            for cp in out_copies(p, cidx):
                cp.wait()

    out_cm = wb(x_cm, idx_1d, tbl_cm)
    return out_cm.reshape(3, n).T


# speedup vs baseline: 81.2372x; 8.9683x over previous
import functools

import jax
import jax.numpy as jnp
from jax import lax
from jax.experimental import pallas as pl
from jax.experimental.pallas import tpu as pltpu
from jax.experimental.pallas import tpu_sc as plsc

_L = 16
_NW = 32


def kernel(x, img_idx, white_balance_parameter):
    n = x.shape[0]
    n_rows = white_balance_parameter.shape[0]
    x_t = x.T
    idx_1d = img_idx.astype(jnp.int32).reshape(n)
    tbl_cm = white_balance_parameter.T.reshape(3 * n_rows)

    per_w = n // _NW
    chunk = 2048
    n_chunks = per_w // chunk

    mesh = plsc.VectorSubcoreMesh(core_axis_name="c", subcore_axis_name="s")

    @functools.partial(
        pl.kernel,
        mesh=mesh,
        compiler_params=pltpu.CompilerParams(needs_layout_passes=False),
        out_type=jax.ShapeDtypeStruct((3, n), jnp.float32),
        scratch_types=[
            pltpu.VMEM((n_rows,), jnp.float32),
            pltpu.VMEM((n_rows,), jnp.float32),
            pltpu.VMEM((n_rows,), jnp.float32),
            pltpu.VMEM((chunk,), jnp.int32),
            pltpu.VMEM((chunk,), jnp.int32),
            pltpu.VMEM((3, chunk), jnp.float32),
            pltpu.VMEM((3, chunk), jnp.float32),
            pltpu.VMEM((3, chunk), jnp.float32),
            pltpu.VMEM((3, chunk), jnp.float32),
            pltpu.SemaphoreType.DMA,
            pltpu.SemaphoreType.DMA,
            pltpu.SemaphoreType.DMA,
            pltpu.SemaphoreType.DMA,
        ],
    )
    def wb(
        x_hbm, idx_hbm, tbl_hbm, out_hbm,
        t0_v, t1_v, t2_v, idx0_v, idx1_v, x0_v, x1_v, o0_v, o1_v,
        sem_in0, sem_in1, sem_out0, sem_out1,
    ):
        wid = lax.axis_index("s") * 2 + lax.axis_index("c")
        ray0 = wid * per_w
        tbls = (t0_v, t1_v, t2_v)
        idx_bufs = (idx0_v, idx1_v)
        x_bufs = (x0_v, x1_v)
        o_bufs = (o0_v, o1_v)
        sems_in = (sem_in0, sem_in1)
        sems_out = (sem_out0, sem_out1)
        for c in range(3):
            pltpu.sync_copy(tbl_hbm.at[pl.ds(c * n_rows, n_rows)], tbls[c])

        def in_copies(p, cidx):
            rbase = ray0 + cidx * chunk
            return [
                pltpu.make_async_copy(
                    idx_hbm.at[pl.ds(rbase, chunk)], idx_bufs[p], sems_in[p]
                ),
                pltpu.make_async_copy(
                    x_hbm.at[pl.ds(0, 3), pl.ds(rbase, chunk)],
                    x_bufs[p],
                    sems_in[p],
                ),
            ]

        def out_copies(p, cidx):
            rbase = ray0 + cidx * chunk
            return [
                pltpu.make_async_copy(
                    o_bufs[p],
                    out_hbm.at[pl.ds(0, 3), pl.ds(rbase, chunk)],
                    sems_out[p],
                )
            ]

        for cp in in_copies(0, 0):
            cp.start()

        def do_chunk(p, cidx):
            @pl.when(cidx + 1 < n_chunks)
            def _():
                for cp in in_copies(1 - p, cidx + 1):
                    cp.start()

            for cp in in_copies(p, cidx):
                cp.wait()

            @pl.when(cidx >= 2)
            def _():
                for cp in out_copies(p, cidx - 2):
                    cp.wait()

            def group_body(g, c2):
                idxv = idx_bufs[p][pl.ds(_L * g, _L)]
                for c in range(3):
                    tv = plsc.load_gather(tbls[c], [idxv])
                    o_bufs[p][c, pl.ds(_L * g, _L)] = (
                        x_bufs[p][c, pl.ds(_L * g, _L)] * tv
                    )
                return c2

            lax.fori_loop(0, chunk // _L, group_body, 0, unroll=8)

            for cp in out_copies(p, cidx):
                cp.start()

        def loop_body(base_cidx, carry):
            do_chunk(0, base_cidx)
            do_chunk(1, base_cidx + 1)
            return carry

        lax.fori_loop(0, n_chunks // 2, lambda i, c: loop_body(2 * i, c), 0)

        for p, cidx in ((0, n_chunks - 2), (1, n_chunks - 1)):
            for cp in out_copies(p, cidx):
                cp.wait()

    out_t = wb(x_t, idx_1d, tbl_cm)
    return out_t.T


# speedup vs baseline: 161.2616x; 1.9851x over previous
import functools

import jax
import jax.numpy as jnp
from jax import lax
from jax.experimental import pallas as pl
from jax.experimental.pallas import tpu as pltpu
from jax.experimental.pallas import tpu_sc as plsc

_L = 16
_NW = 32


def kernel(x, img_idx, white_balance_parameter):
    n = x.shape[0]
    n_rows = white_balance_parameter.shape[0]
    x_t = x.T
    idx_1d = img_idx.astype(jnp.int32).reshape(n)
    tbl_cm = white_balance_parameter.T.reshape(3 * n_rows)

    per_w = n // _NW
    chunk = 2048
    n_chunks = per_w // chunk

    mesh = plsc.VectorSubcoreMesh(core_axis_name="c", subcore_axis_name="s")

    @functools.partial(
        pl.kernel,
        mesh=mesh,
        compiler_params=pltpu.CompilerParams(needs_layout_passes=False),
        out_type=jax.ShapeDtypeStruct((3, n), jnp.float32),
        scratch_types=[
            pltpu.VMEM((n_rows,), jnp.float32),
            pltpu.VMEM((n_rows,), jnp.float32),
            pltpu.VMEM((n_rows,), jnp.float32),
            pltpu.VMEM((chunk,), jnp.int32),
            pltpu.VMEM((chunk,), jnp.int32),
            pltpu.VMEM((3, chunk), jnp.float32),
            pltpu.VMEM((3, chunk), jnp.float32),
            pltpu.VMEM((3, chunk), jnp.float32),
            pltpu.VMEM((3, chunk), jnp.float32),
            pltpu.SemaphoreType.DMA,
            pltpu.SemaphoreType.DMA,
            pltpu.SemaphoreType.DMA,
            pltpu.SemaphoreType.DMA,
        ],
    )
    def wb(
        x_hbm, idx_hbm, tbl_hbm, out_hbm,
        t0_v, t1_v, t2_v, idx0_v, idx1_v, x0_v, x1_v, o0_v, o1_v,
        sem_in0, sem_in1, sem_out0, sem_out1,
    ):
        wid = lax.axis_index("s") * 2 + lax.axis_index("c")
        ray0 = wid * per_w
        tbls = (t0_v, t1_v, t2_v)
        idx_bufs = (idx0_v, idx1_v)
        x_bufs = (x0_v, x1_v)
        o_bufs = (o0_v, o1_v)
        sems_in = (sem_in0, sem_in1)
        sems_out = (sem_out0, sem_out1)
        for c in range(3):
            pltpu.sync_copy(tbl_hbm.at[pl.ds(c * n_rows, n_rows)], tbls[c])

        def in_copies(p, cidx):
            rbase = ray0 + cidx * chunk
            return [
                pltpu.make_async_copy(
                    idx_hbm.at[pl.ds(rbase, chunk)], idx_bufs[p], sems_in[p]
                ),
                pltpu.make_async_copy(
                    x_hbm.at[pl.ds(0, 3), pl.ds(rbase, chunk)],
                    x_bufs[p],
                    sems_in[p],
                ),
            ]

        def out_copies(p, cidx):
            rbase = ray0 + cidx * chunk
            return [
                pltpu.make_async_copy(
                    o_bufs[p],
                    out_hbm.at[pl.ds(0, 3), pl.ds(rbase, chunk)],
                    sems_out[p],
                )
            ]

        for cp in in_copies(0, 0):
            cp.start()

        def do_chunk(p, cidx):
            @pl.when(cidx + 1 < n_chunks)
            def _():
                for cp in in_copies(1 - p, cidx + 1):
                    cp.start()

            for cp in in_copies(p, cidx):
                cp.wait()

            @pl.when(cidx >= 2)
            def _():
                for cp in out_copies(p, cidx - 2):
                    cp.wait()

            @plsc.parallel_loop(0, chunk // _L, unroll=8)
            def _(g):
                idxv = idx_bufs[p][pl.ds(_L * g, _L)]
                xs = [x_bufs[p][c, pl.ds(_L * g, _L)] for c in range(3)]
                tvs = [plsc.load_gather(tbls[c], [idxv]) for c in range(3)]
                for c in range(3):
                    o_bufs[p][c, pl.ds(_L * g, _L)] = xs[c] * tvs[c]

            for cp in out_copies(p, cidx):
                cp.start()

        def loop_body(base_cidx, carry):
            do_chunk(0, base_cidx)
            do_chunk(1, base_cidx + 1)
            return carry

        lax.fori_loop(0, n_chunks // 2, lambda i, c: loop_body(2 * i, c), 0)

        for p, cidx in ((0, n_chunks - 2), (1, n_chunks - 1)):
            for cp in out_copies(p, cidx):
                cp.wait()

    out_t = wb(x_t, idx_1d, tbl_cm)
    return out_t.T


# speedup vs baseline: 165.2247x; 1.0246x over previous
import functools

import jax
import jax.numpy as jnp
from jax import lax
from jax.experimental import pallas as pl
from jax.experimental.pallas import tpu as pltpu
from jax.experimental.pallas import tpu_sc as plsc

_L = 16
_NW = 32


def kernel(x, img_idx, white_balance_parameter):
    n = x.shape[0]
    n_rows = white_balance_parameter.shape[0]
    x_t = x.T
    idx_1d = img_idx.astype(jnp.int32).reshape(n)
    tbl_cm = white_balance_parameter.T.reshape(3 * n_rows)

    per_w = n // _NW
    chunk = 4096
    n_chunks = per_w // chunk

    mesh = plsc.VectorSubcoreMesh(core_axis_name="c", subcore_axis_name="s")

    @functools.partial(
        pl.kernel,
        mesh=mesh,
        compiler_params=pltpu.CompilerParams(needs_layout_passes=False),
        out_type=jax.ShapeDtypeStruct((3, n), jnp.float32),
        scratch_types=[
            pltpu.VMEM((n_rows,), jnp.float32),
            pltpu.VMEM((n_rows,), jnp.float32),
            pltpu.VMEM((n_rows,), jnp.float32),
            pltpu.VMEM((chunk,), jnp.int32),
            pltpu.VMEM((chunk,), jnp.int32),
            pltpu.VMEM((8, chunk), jnp.float32),
            pltpu.VMEM((8, chunk), jnp.float32),
            pltpu.SemaphoreType.DMA,
            pltpu.SemaphoreType.DMA,
            pltpu.SemaphoreType.DMA,
            pltpu.SemaphoreType.DMA,
        ],
    )
    def wb(
        x_hbm, idx_hbm, tbl_hbm, out_hbm,
        t0_v, t1_v, t2_v, idx0_v, idx1_v, b0_v, b1_v,
        sem_in0, sem_in1, sem_out0, sem_out1,
    ):
        wid = lax.axis_index("s") * 2 + lax.axis_index("c")
        ray0 = wid * per_w
        tbls = (t0_v, t1_v, t2_v)
        idx_bufs = (idx0_v, idx1_v)
        bufs = (b0_v, b1_v)
        sems_in = (sem_in0, sem_in1)
        sems_out = (sem_out0, sem_out1)
        for c in range(3):
            pltpu.sync_copy(tbl_hbm.at[pl.ds(c * n_rows, n_rows)], tbls[c])

        def in_copies(p, cidx):
            rbase = ray0 + cidx * chunk
            return [
                pltpu.make_async_copy(
                    idx_hbm.at[pl.ds(rbase, chunk)], idx_bufs[p], sems_in[p]
                ),
                pltpu.make_async_copy(
                    x_hbm.at[pl.ds(0, 3), pl.ds(rbase, chunk)],
                    bufs[p].at[pl.ds(0, 3)],
                    sems_in[p],
                ),
            ]

        def out_copies(p, cidx):
            rbase = ray0 + cidx * chunk
            return [
                pltpu.make_async_copy(
                    bufs[p].at[pl.ds(4, 3)],
                    out_hbm.at[pl.ds(0, 3), pl.ds(rbase, chunk)],
                    sems_out[p],
                )
            ]

        for cp in in_copies(0, 0):
            cp.start()

        def do_chunk(p, cidx):
            @pl.when(cidx + 1 < n_chunks)
            def _():
                for cp in in_copies(1 - p, cidx + 1):
                    cp.start()

            for cp in in_copies(p, cidx):
                cp.wait()

            @pl.when(cidx >= 2)
            def _():
                for cp in out_copies(p, cidx - 2):
                    cp.wait()

            @plsc.parallel_loop(0, chunk // _L, unroll=8)
            def _(g):
                idxv = idx_bufs[p][pl.ds(_L * g, _L)]
                xs = [bufs[p][c, pl.ds(_L * g, _L)] for c in range(3)]
                tvs = [plsc.load_gather(tbls[c], [idxv]) for c in range(3)]
                for c in range(3):
                    bufs[p][4 + c, pl.ds(_L * g, _L)] = xs[c] * tvs[c]

            for cp in out_copies(p, cidx):
                cp.start()

        def loop_body(base_cidx, carry):
            do_chunk(0, base_cidx)
            do_chunk(1, base_cidx + 1)
            return carry

        lax.fori_loop(0, n_chunks // 2, lambda i, c: loop_body(2 * i, c), 0)

        for p, cidx in ((0, n_chunks - 2), (1, n_chunks - 1)):
            for cp in out_copies(p, cidx):
                cp.wait()

    out_t = wb(x_t, idx_1d, tbl_cm)
    return out_t.T
